# baseline (speedup 1.0000x reference)
import functools

import jax
import jax.numpy as jnp
from jax import lax
from jax.experimental import pallas as pl
from jax.experimental.pallas import tpu as pltpu
from jax.experimental.pallas import tpu_sc as plsc


def _sc_gather(B, H, W, D, N, T):
    info = plsc.get_sparse_core_info()
    NC, NS, L = info.num_cores, info.num_subcores, info.num_lanes
    NW = NC * NS
    PAIRS = B * N
    ppw = PAIRS // NW
    CP = 2
    ROWS = CP * H
    nchunks = ppw // CP
    scale = float(W) / float(T)

    mesh = plsc.VectorSubcoreMesh(core_axis_name="c", subcore_axis_name="s")

    @functools.partial(
        pl.kernel,
        mesh=mesh,
        out_type=jax.ShapeDtypeStruct((PAIRS * H, D), jnp.float32),
        scratch_types=[
            pltpu.VMEM((ppw,), jnp.int32),
            pltpu.VMEM((2, ROWS), jnp.int32),
            pltpu.VMEM((2, ROWS, D), jnp.float32),
            pltpu.SemaphoreType.DMA,
            pltpu.SemaphoreType.DMA,
        ],
        compiler_params=pltpu.CompilerParams(use_tc_tiling_on_sc=False),
    )
    def k(swin_hbm, fire_hbm, out_hbm, fw_v, idx_v, rows_v, gsem0, gsem1):
        wid = lax.axis_index("s") * NC + lax.axis_index("c")
        pair_base = wid * ppw
        b = pair_base // N
        hbase = b * (H * W)
        gsems = (gsem0, gsem1)
        pltpu.sync_copy(fire_hbm.at[pl.ds(pair_base, ppw)], fw_v)
        for v in range(ppw // L):
            f = fw_v[pl.ds(v * L, L)]
            w = (f.astype(jnp.float32) * scale).astype(jnp.int32)
            fw_v[pl.ds(v * L, L)] = jnp.clip(w, 0, W - 1)

        def build_and_fire(c):
            s = c % 2
            fvec = fw_v[pl.ds((c * CP // L) * L, L)]
            for p in range(CP):
                pli = c * CP + p
                fwp = fvec[pli % L]
                for g in range(H // L):
                    hvec = lax.iota(jnp.int32, L) + (g * L)
                    idx_v[s, pl.ds(p * H + g * L, L)] = hbase + hvec * W + fwp
            return pltpu.async_copy(swin_hbm.at[idx_v.at[s]], rows_v.at[s], gsems[s])

        copies = [build_and_fire(0)]
        for c in range(nchunks):
            if c + 1 < nchunks:
                copies.append(build_and_fire(c + 1))
            copies[c].wait()
            pltpu.sync_copy(
                rows_v.at[c % 2], out_hbm.at[pl.ds((pair_base + c * CP) * H, ROWS)]
            )

    return k


def _tc_project(G, PAIRS, H, D, P, DP, DM):
    def body(g_ref, p_ref, wsw_ref, bsw_ref, wp_ref, bp_ref, o_ref):
        sw = jnp.dot(g_ref[...], wsw_ref[...], preferred_element_type=jnp.float32)
        o_ref[:, P:, :] = (sw + bsw_ref[...]).reshape(G, H, DM)
        pt = jnp.dot(p_ref[...], wp_ref[...], preferred_element_type=jnp.float32)
        o_ref[:, :P, :] = (pt + bp_ref[...]).reshape(G, P, DM)

    return pl.pallas_call(
        body,
        grid=(PAIRS // G,),
        in_specs=[
            pl.BlockSpec((G * H, D), lambda i: (i, 0)),
            pl.BlockSpec((G * P, DP), lambda i: (i, 0)),
            pl.BlockSpec((D, DM), lambda i: (0, 0)),
            pl.BlockSpec((1, DM), lambda i: (0, 0)),
            pl.BlockSpec((DP, DM), lambda i: (0, 0)),
            pl.BlockSpec((1, DM), lambda i: (0, 0)),
        ],
        out_specs=pl.BlockSpec((G, P + H, DM), lambda i: (i, 0, 0)),
        out_shape=jax.ShapeDtypeStruct((PAIRS, P + H, DM), jnp.float32),
        compiler_params=pltpu.CompilerParams(
            dimension_semantics=("arbitrary",),
        ),
    )


def kernel(fire_signal, swin_2d, fire_frames, pitch_tokens, W_pitch, b_pitch, W_sw, b_sw):
    B, H, W, D = swin_2d.shape
    T = fire_signal.shape[1]
    N = fire_frames.shape[1]
    P, DP = pitch_tokens.shape[2], pitch_tokens.shape[3]
    DM = W_pitch.shape[1]
    PAIRS = B * N

    swin_flat = swin_2d.reshape(B * H * W, D)
    fire_flat = fire_frames.reshape(PAIRS)
    gathered = _sc_gather(B, H, W, D, N, T)(swin_flat, fire_flat)

    out = _tc_project(16, PAIRS, H, D, P, DP, DM)(
        gathered,
        pitch_tokens.reshape(PAIRS * P, DP),
        W_sw,
        b_sw.reshape(1, DM),
        W_pitch,
        b_pitch.reshape(1, DM),
    )
    return out.reshape(B, N, P + H, DM)


# speedup vs baseline: 3.4760x; 3.4760x over previous
import functools

import jax
import jax.numpy as jnp
from jax import lax
from jax.experimental import pallas as pl
from jax.experimental.pallas import tpu as pltpu
from jax.experimental.pallas import tpu_sc as plsc

_NBUF = 6
_PRE = 3


def _sc_gather(B, H, W, D, N, T):
    info = plsc.get_sparse_core_info()
    NC, NS, L = info.num_cores, info.num_subcores, info.num_lanes
    NW = NC * NS
    PAIRS = B * N
    ppw = PAIRS // NW
    scale = float(W) / float(T)

    mesh = plsc.VectorSubcoreMesh(core_axis_name="c", subcore_axis_name="s")

    @functools.partial(
        pl.kernel,
        mesh=mesh,
        out_type=jax.ShapeDtypeStruct((PAIRS * H, D), jnp.float32),
        scratch_types=[
            pltpu.VMEM((ppw,), jnp.int32),
            pltpu.VMEM((_NBUF, H, D), jnp.float32),
            pltpu.SemaphoreType.DMA((_NBUF,)),
            pltpu.SemaphoreType.DMA((_NBUF,)),
        ],
    )
    def k(swin_hbm, fire_hbm, out_hbm, fw_v, rows_v, gsem, osem):
        wid = lax.axis_index("s") * NC + lax.axis_index("c")
        pair_base = wid * ppw
        b = pair_base // N
        pltpu.sync_copy(fire_hbm.at[pl.ds(pair_base, ppw)], fw_v)
        for v in range(ppw // L):
            f = fw_v[pl.ds(v * L, L)]
            w = (f.astype(jnp.float32) * scale).astype(jnp.int32)
            fw_v[pl.ds(v * L, L)] = jnp.clip(w, 0, W - 1)

        def fire_gather(p):
            s = p % _NBUF
            fvec = fw_v[pl.ds((p // L) * L, L)]
            wp = fvec[p % L]
            return pltpu.async_copy(swin_hbm.at[b, :, wp, :], rows_v.at[s], gsem.at[s])

        gc = {}
        oc = {}
        for p in range(_PRE):
            gc[p] = fire_gather(p)
        for p in range(ppw):
            q = p + _PRE
            if q < ppw:
                if q >= _NBUF:
                    oc[q - _NBUF].wait()
                gc[q] = fire_gather(q)
            gc[p].wait()
            oc[p] = pltpu.async_copy(
                rows_v.at[p % _NBUF], out_hbm.at[pl.ds((pair_base + p) * H, H)],
                osem.at[p % _NBUF],
            )
        for p in range(ppw - _NBUF, ppw):
            oc[p].wait()

    return k


def _tc_project(G, B, N, H, D, P, DP, DM):
    def body(g_ref, p_ref, wsw_ref, bsw_ref, wp_ref, bp_ref, o_ref):
        sw = jnp.dot(g_ref[...], wsw_ref[...], preferred_element_type=jnp.float32)
        o_ref[0, :, P:, :] = (sw + bsw_ref[...]).reshape(G, H, DM)
        pt = jnp.dot(p_ref[...].reshape(G * P, DP), wp_ref[...],
                     preferred_element_type=jnp.float32)
        o_ref[0, :, :P, :] = (pt + bp_ref[...]).reshape(G, P, DM)

    return pl.pallas_call(
        body,
        grid=(B, N // G),
        in_specs=[
            pl.BlockSpec((G * H, D), lambda i, j: (i * (N // G) + j, 0)),
            pl.BlockSpec((1, G, P, DP), lambda i, j: (i, j, 0, 0)),
            pl.BlockSpec((D, DM), lambda i, j: (0, 0)),
            pl.BlockSpec((1, DM), lambda i, j: (0, 0)),
            pl.BlockSpec((DP, DM), lambda i, j: (0, 0)),
            pl.BlockSpec((1, DM), lambda i, j: (0, 0)),
        ],
        out_specs=pl.BlockSpec((1, G, P + H, DM), lambda i, j: (i, j, 0, 0)),
        out_shape=jax.ShapeDtypeStruct((B, N, P + H, DM), jnp.float32),
        compiler_params=pltpu.CompilerParams(
            dimension_semantics=("arbitrary", "arbitrary"),
        ),
    )


def kernel(fire_signal, swin_2d, fire_frames, pitch_tokens, W_pitch, b_pitch, W_sw, b_sw):
    B, H, W, D = swin_2d.shape
    T = fire_signal.shape[1]
    N = fire_frames.shape[1]
    P, DP = pitch_tokens.shape[2], pitch_tokens.shape[3]
    DM = W_pitch.shape[1]

    fire_flat = fire_frames.reshape(B * N)
    gathered = _sc_gather(B, H, W, D, N, T)(swin_2d, fire_flat)

    return _tc_project(16, B, N, H, D, P, DP, DM)(
        gathered,
        pitch_tokens,
        W_sw,
        b_sw.reshape(1, DM),
        W_pitch,
        b_pitch.reshape(1, DM),
    )


# speedup vs baseline: 5.4200x; 1.5593x over previous
import functools

import jax
import jax.numpy as jnp
from jax import lax
from jax.experimental import pallas as pl
from jax.experimental.pallas import tpu as pltpu
from jax.experimental.pallas import tpu_sc as plsc

_HB = 8


def _sc_onehot(B, W, N, T):
    info = plsc.get_sparse_core_info()
    NC, NS, L = info.num_cores, info.num_subcores, info.num_lanes
    scale = float(W) / float(T)

    mesh = plsc.VectorSubcoreMesh(core_axis_name="c", subcore_axis_name="s")

    @functools.partial(
        pl.kernel,
        mesh=mesh,
        out_type=jax.ShapeDtypeStruct((B, W * N), jnp.float32),
        scratch_types=[
            pltpu.VMEM((N,), jnp.int32),
            pltpu.VMEM((W * N,), jnp.float32),
            pltpu.SemaphoreType.DMA,
        ],
        compiler_params=pltpu.CompilerParams(needs_layout_passes=False),
    )
    def k(fire_hbm, s_hbm, fw_v, s_v, sem):
        wid = lax.axis_index("s") * NC + lax.axis_index("c")

        @pl.when(wid < B)
        def _():
            pltpu.sync_copy(fire_hbm.at[pl.ds(wid * N, N)], fw_v)
            for v in range(N // L):
                f = fw_v[pl.ds(v * L, L)]
                w = (f.astype(jnp.float32) * scale).astype(jnp.int32)
                fw_v[pl.ds(v * L, L)] = jnp.clip(w, 0, W - 1)

            zero = jnp.zeros((L,), jnp.float32)

            def zrow(i, carry):
                s_v[pl.ds(i * L, L)] = zero
                return carry

            lax.fori_loop(0, W * N // L, zrow, 0)
            one = jnp.ones((L,), jnp.float32)
            for c in range(N // L):
                fv = fw_v[pl.ds(c * L, L)]
                idx = fv * N + (lax.iota(jnp.int32, L) + c * L)
                plsc.store_scatter(s_v, [idx], one)
            pltpu.sync_copy(s_v, s_hbm.at[wid])

    return k


def _tc_fused(B, H, W, D, N, P, DP, DM):
    def body(x_ref, s_ref, p_ref, wsw_ref, bsw_ref, wp_ref, bp_ref, o_ref):
        j = pl.program_id(1)

        @pl.when(j == 0)
        def _():
            pt = jnp.dot(p_ref[0].reshape(N * P, DP), wp_ref[...],
                         preferred_element_type=jnp.float32)
            o_ref[0, :, :P, :] = (pt + bp_ref[...]).reshape(N, P, DM)

        S_b = s_ref[0]
        for h in range(_HB):
            Xb = x_ref[0, h].astype(jnp.bfloat16)
            G = lax.dot_general(S_b, Xb, (((0,), (1,)), ((), ())),
                                preferred_element_type=jnp.float32)
            R = jnp.dot(G.astype(jnp.bfloat16), wsw_ref[...],
                        preferred_element_type=jnp.float32)
            o_ref[0, :, P + j * _HB + h, :] = R + bsw_ref[...]

    return pl.pallas_call(
        body,
        grid=(B, H // _HB),
        in_specs=[
            pl.BlockSpec((1, _HB, D, W), lambda b, j: (b, j, 0, 0)),
            pl.BlockSpec((1, W, N), lambda b, j: (b, 0, 0)),
            pl.BlockSpec((1, N, P, DP), lambda b, j: (b, 0, 0, 0)),
            pl.BlockSpec((D, DM), lambda b, j: (0, 0)),
            pl.BlockSpec((1, DM), lambda b, j: (0, 0)),
            pl.BlockSpec((DP, DM), lambda b, j: (0, 0)),
            pl.BlockSpec((1, DM), lambda b, j: (0, 0)),
        ],
        out_specs=pl.BlockSpec((1, N, P + H, DM), lambda b, j: (b, 0, 0, 0)),
        out_shape=jax.ShapeDtypeStruct((B, N, P + H, DM), jnp.float32),
        compiler_params=pltpu.CompilerParams(
            dimension_semantics=("arbitrary", "arbitrary"),
        ),
    )


def kernel(fire_signal, swin_2d, fire_frames, pitch_tokens, W_pitch, b_pitch, W_sw, b_sw):
    B, H, W, D = swin_2d.shape
    T = fire_signal.shape[1]
    N = fire_frames.shape[1]
    P, DP = pitch_tokens.shape[2], pitch_tokens.shape[3]
    DM = W_pitch.shape[1]

    fire_flat = fire_frames.reshape(B * N)
    sel = _sc_onehot(B, W, N, T)(fire_flat).reshape(B, W, N).astype(jnp.bfloat16)
    swin_t = jnp.transpose(swin_2d, (0, 1, 3, 2))

    return _tc_fused(B, H, W, D, N, P, DP, DM)(
        swin_t,
        sel,
        pitch_tokens,
        W_sw.astype(jnp.bfloat16),
        b_sw.reshape(1, DM),
        W_pitch,
        b_pitch.reshape(1, DM),
    )


# speedup vs baseline: 5.8665x; 1.0824x over previous
import functools

import jax
import jax.numpy as jnp
from jax import lax
from jax.experimental import pallas as pl
from jax.experimental.pallas import tpu as pltpu
from jax.experimental.pallas import tpu_sc as plsc

_HB = 16


def _sc_onehot(B, W, N, T):
    info = plsc.get_sparse_core_info()
    NC, NS, L = info.num_cores, info.num_subcores, info.num_lanes
    scale = float(W) / float(T)

    mesh = plsc.VectorSubcoreMesh(core_axis_name="c", subcore_axis_name="s")

    @functools.partial(
        pl.kernel,
        mesh=mesh,
        out_type=jax.ShapeDtypeStruct((B, W * N), jnp.float32),
        scratch_types=[
            pltpu.VMEM((N,), jnp.int32),
            pltpu.VMEM((W * N,), jnp.float32),
            pltpu.SemaphoreType.DMA,
        ],
        compiler_params=pltpu.CompilerParams(needs_layout_passes=False),
    )
    def k(fire_hbm, s_hbm, fw_v, s_v, sem):
        wid = lax.axis_index("s") * NC + lax.axis_index("c")

        @pl.when(wid < B)
        def _():
            pltpu.sync_copy(fire_hbm.at[pl.ds(wid * N, N)], fw_v)
            for v in range(N // L):
                f = fw_v[pl.ds(v * L, L)]
                w = (f.astype(jnp.float32) * scale).astype(jnp.int32)
                fw_v[pl.ds(v * L, L)] = jnp.clip(w, 0, W - 1)

            zero = jnp.zeros((L,), jnp.float32)

            def zrow(i, carry):
                s_v[pl.ds(i * L, L)] = zero
                return carry

            lax.fori_loop(0, W * N // L, zrow, 0)
            one = jnp.ones((L,), jnp.float32)
            for c in range(N // L):
                fv = fw_v[pl.ds(c * L, L)]
                idx = fv * N + (lax.iota(jnp.int32, L) + c * L)
                plsc.store_scatter(s_v, [idx], one)
            pltpu.sync_copy(s_v, s_hbm.at[wid])

    return k


def _tc_fused(B, H, W, D, N, P, DP, DM):
    def body(x_ref, s_ref, p_ref, wsw_ref, bsw_ref, wp_ref, bp_ref, o_ref):
        j = pl.program_id(1)

        @pl.when(j == 0)
        def _():
            pt = jnp.dot(p_ref[0].reshape(N * P, DP), wp_ref[...],
                         preferred_element_type=jnp.float32)
            o_ref[0, :, :P, :] = (pt + bp_ref[...]).reshape(N, P, DM)

        S_b = s_ref[0]
        bias = bsw_ref[...]
        for h in range(_HB):
            G = lax.dot_general(S_b, x_ref[0, h], (((0,), (1,)), ((), ())),
                                preferred_element_type=jnp.float32,
                                precision=lax.Precision.DEFAULT)
            R = jnp.dot(G, wsw_ref[...], preferred_element_type=jnp.float32,
                        precision=lax.Precision.DEFAULT)
            o_ref[0, :, P + j * _HB + h, :] = R + bias

    return pl.pallas_call(
        body,
        grid=(B, H // _HB),
        in_specs=[
            pl.BlockSpec((1, _HB, D, W), lambda b, j: (b, j, 0, 0)),
            pl.BlockSpec((1, W, N), lambda b, j: (b, 0, 0)),
            pl.BlockSpec((1, N, P, DP), lambda b, j: (b, 0, 0, 0)),
            pl.BlockSpec((D, DM), lambda b, j: (0, 0)),
            pl.BlockSpec((1, DM), lambda b, j: (0, 0)),
            pl.BlockSpec((DP, DM), lambda b, j: (0, 0)),
            pl.BlockSpec((1, DM), lambda b, j: (0, 0)),
        ],
        out_specs=pl.BlockSpec((1, N, P + H, DM), lambda b, j: (b, 0, 0, 0)),
        out_shape=jax.ShapeDtypeStruct((B, N, P + H, DM), jnp.float32),
        compiler_params=pltpu.CompilerParams(
            dimension_semantics=("arbitrary", "arbitrary"),
        ),
    )


def kernel(fire_signal, swin_2d, fire_frames, pitch_tokens, W_pitch, b_pitch, W_sw, b_sw):
    B, H, W, D = swin_2d.shape
    T = fire_signal.shape[1]
    N = fire_frames.shape[1]
    P, DP = pitch_tokens.shape[2], pitch_tokens.shape[3]
    DM = W_pitch.shape[1]

    fire_flat = fire_frames.reshape(B * N)
    sel = _sc_onehot(B, W, N, T)(fire_flat).reshape(B, W, N).astype(jnp.bfloat16)
    swin_t = jnp.transpose(swin_2d, (0, 1, 3, 2))

    return _tc_fused(B, H, W, D, N, P, DP, DM)(
        swin_t,
        sel,
        pitch_tokens,
        W_sw.astype(jnp.bfloat16),
        b_sw.reshape(1, DM),
        W_pitch,
        b_pitch.reshape(1, DM),
    )


# speedup vs baseline: 6.2286x; 1.0617x over previous
import functools

import jax
import jax.numpy as jnp
from jax import lax
from jax.experimental import pallas as pl
from jax.experimental.pallas import tpu as pltpu
from jax.experimental.pallas import tpu_sc as plsc

_HB = 16


def _sc_onehot(B, W, N, T):
    info = plsc.get_sparse_core_info()
    NC, NS, L = info.num_cores, info.num_subcores, info.num_lanes
    NW = NC * NS
    QS = NW // B
    WQ = (W + QS - 1) // QS
    Wp = WQ * QS
    scale = float(W) / float(T)

    mesh = plsc.VectorSubcoreMesh(core_axis_name="c", subcore_axis_name="s")

    @functools.partial(
        pl.kernel,
        mesh=mesh,
        out_type=jax.ShapeDtypeStruct((B, Wp * N), jnp.float32),
        scratch_types=[
            pltpu.VMEM((N,), jnp.int32),
            pltpu.VMEM((WQ * N,), jnp.float32),
            pltpu.SemaphoreType.DMA,
        ],
        compiler_params=pltpu.CompilerParams(needs_layout_passes=False),
    )
    def k(fire_hbm, s_hbm, fw_v, s_v, sem):
        wid = lax.axis_index("s") * NC + lax.axis_index("c")
        b = wid // QS
        lo = (wid % QS) * WQ
        pltpu.sync_copy(fire_hbm.at[pl.ds(b * N, N)], fw_v)
        for v in range(N // L):
            f = fw_v[pl.ds(v * L, L)]
            w = (f.astype(jnp.float32) * scale).astype(jnp.int32)
            fw_v[pl.ds(v * L, L)] = jnp.clip(w, 0, W - 1)

        zero = jnp.zeros((L,), jnp.float32)

        def zrow(i, carry):
            s_v[pl.ds(i * L, L)] = zero
            return carry

        lax.fori_loop(0, WQ * N // L, zrow, 0)
        one = jnp.ones((L,), jnp.float32)
        for c in range(N // L):
            fv = fw_v[pl.ds(c * L, L)]
            rel = jnp.clip(fv - lo, 0, WQ - 1)
            idx = rel * N + (lax.iota(jnp.int32, L) + c * L)
            msk = (fv >= lo) & (fv < lo + WQ)
            plsc.store_scatter(s_v, [idx], one, mask=msk)
        pltpu.sync_copy(s_v, s_hbm.at[b, pl.ds(lo * N, WQ * N)])

    return k


def _tc_fused(B, H, W, D, N, P, DP, DM):
    def body(x_ref, s_ref, p_ref, wsw_ref, bsw_ref, wp_ref, bp_ref, o_ref):
        j = pl.program_id(1)

        @pl.when(j == 0)
        def _():
            pt = jnp.dot(p_ref[0].reshape(N * P, DP), wp_ref[...],
                         preferred_element_type=jnp.float32)
            o_ref[0, :, :P, :] = (pt + bp_ref[...]).reshape(N, P, DM)

        S_b = s_ref[0]
        bias = bsw_ref[...]
        for h in range(_HB):
            G = lax.dot_general(S_b, x_ref[0, h], (((0,), (1,)), ((), ())),
                                preferred_element_type=jnp.float32,
                                precision=lax.Precision.DEFAULT)
            R = jnp.dot(G, wsw_ref[...], preferred_element_type=jnp.float32,
                        precision=lax.Precision.DEFAULT)
            o_ref[0, :, P + j * _HB + h, :] = R + bias

    return pl.pallas_call(
        body,
        grid=(B, H // _HB),
        in_specs=[
            pl.BlockSpec((1, _HB, D, W), lambda b, j: (b, j, 0, 0)),
            pl.BlockSpec((1, W, N), lambda b, j: (b, 0, 0)),
            pl.BlockSpec((1, N, P, DP), lambda b, j: (b, 0, 0, 0)),
            pl.BlockSpec((D, DM), lambda b, j: (0, 0)),
            pl.BlockSpec((1, DM), lambda b, j: (0, 0)),
            pl.BlockSpec((DP, DM), lambda b, j: (0, 0)),
            pl.BlockSpec((1, DM), lambda b, j: (0, 0)),
        ],
        out_specs=pl.BlockSpec((1, N, P + H, DM), lambda b, j: (b, 0, 0, 0)),
        out_shape=jax.ShapeDtypeStruct((B, N, P + H, DM), jnp.float32),
        compiler_params=pltpu.CompilerParams(
            dimension_semantics=("arbitrary", "arbitrary"),
        ),
    )


def kernel(fire_signal, swin_2d, fire_frames, pitch_tokens, W_pitch, b_pitch, W_sw, b_sw):
    B, H, W, D = swin_2d.shape
    T = fire_signal.shape[1]
    N = fire_frames.shape[1]
    P, DP = pitch_tokens.shape[2], pitch_tokens.shape[3]
    DM = W_pitch.shape[1]

    fire_flat = fire_frames.reshape(B * N)
    sel = _sc_onehot(B, W, N, T)(fire_flat)[:, : W * N]
    sel = sel.reshape(B, W, N).astype(jnp.bfloat16)
    swin_t = jnp.transpose(swin_2d, (0, 1, 3, 2))

    return _tc_fused(B, H, W, D, N, P, DP, DM)(
        swin_t,
        sel,
        pitch_tokens,
        W_sw.astype(jnp.bfloat16),
        b_sw.reshape(1, DM),
        W_pitch,
        b_pitch.reshape(1, DM),
    )


# speedup vs baseline: 6.2620x; 1.0054x over previous
import functools

import jax
import jax.numpy as jnp
from jax import lax
from jax.experimental import pallas as pl
from jax.experimental.pallas import tpu as pltpu
from jax.experimental.pallas import tpu_sc as plsc

_HB = 16


def _sc_onehot(B, W, N, T):
    info = plsc.get_sparse_core_info()
    NC, NS, L = info.num_cores, info.num_subcores, info.num_lanes
    NW = NC * NS
    QS = NW // B
    WQ = (W + QS - 1) // QS
    Wp = WQ * QS
    scale = float(W) / float(T)

    mesh = plsc.VectorSubcoreMesh(core_axis_name="c", subcore_axis_name="s")

    @functools.partial(
        pl.kernel,
        mesh=mesh,
        out_type=jax.ShapeDtypeStruct((B, Wp * N), jnp.float32),
        scratch_types=[
            pltpu.VMEM((N,), jnp.int32),
            pltpu.VMEM((WQ * N,), jnp.float32),
            pltpu.SemaphoreType.DMA,
        ],
        compiler_params=pltpu.CompilerParams(needs_layout_passes=False),
    )
    def k(fire_hbm, s_hbm, fw_v, s_v, sem):
        wid = lax.axis_index("s") * NC + lax.axis_index("c")
        b = wid // QS
        lo = (wid % QS) * WQ
        pltpu.sync_copy(fire_hbm.at[pl.ds(b * N, N)], fw_v)
        for v in range(N // L):
            f = fw_v[pl.ds(v * L, L)]
            w = (f.astype(jnp.float32) * scale).astype(jnp.int32)
            fw_v[pl.ds(v * L, L)] = jnp.clip(w, 0, W - 1)

        zero = jnp.zeros((L,), jnp.float32)

        def zrow(i, carry):
            s_v[pl.ds(i * L, L)] = zero
            return carry

        lax.fori_loop(0, WQ * N // L, zrow, 0)
        one = jnp.ones((L,), jnp.float32)
        for c in range(N // L):
            fv = fw_v[pl.ds(c * L, L)]
            rel = jnp.clip(fv - lo, 0, WQ - 1)
            idx = rel * N + (lax.iota(jnp.int32, L) + c * L)
            msk = (fv >= lo) & (fv < lo + WQ)
            plsc.store_scatter(s_v, [idx], one, mask=msk)
        pltpu.sync_copy(s_v, s_hbm.at[b, pl.ds(lo * N, WQ * N)])

    return k


def _tc_fused(B, H, W, D, N, P, DP, DM):
    def body(x_ref, s_ref, p_ref, wsw_ref, bsw_ref, wp_ref, bp_ref, o_ref):
        j = pl.program_id(1)

        @pl.when(j == 0)
        def _():
            pt = jnp.dot(p_ref[0].reshape(N * P, DP), wp_ref[...],
                         preferred_element_type=jnp.float32)
            o_ref[0, :, :P, :] = (pt + bp_ref[...]).reshape(N, P, DM)

        S_b = s_ref[0]
        bias = bsw_ref[...]
        for h in range(_HB):
            G = lax.dot_general(S_b, x_ref[0, h], (((0,), (1,)), ((), ())),
                                preferred_element_type=jnp.float32,
                                precision=lax.Precision.DEFAULT)
            R = jnp.dot(G, wsw_ref[...], preferred_element_type=jnp.float32,
                        precision=lax.Precision.DEFAULT)
            o_ref[0, :, P + j * _HB + h, :] = R + bias

    return pl.pallas_call(
        body,
        grid=(B, H // _HB),
        in_specs=[
            pl.BlockSpec((1, _HB, D, W), lambda b, j: (b, j, 0, 0)),
            pl.BlockSpec((1, W, N), lambda b, j: (b, 0, 0)),
            pl.BlockSpec((1, N, P, DP), lambda b, j: (b, 0, 0, 0)),
            pl.BlockSpec((D, DM), lambda b, j: (0, 0)),
            pl.BlockSpec((1, DM), lambda b, j: (0, 0)),
            pl.BlockSpec((DP, DM), lambda b, j: (0, 0)),
            pl.BlockSpec((1, DM), lambda b, j: (0, 0)),
        ],
        out_specs=pl.BlockSpec((1, N, P + H, DM), lambda b, j: (b, 0, 0, 0)),
        out_shape=jax.ShapeDtypeStruct((B, N, P + H, DM), jnp.float32),
        compiler_params=pltpu.CompilerParams(
            dimension_semantics=("arbitrary", "arbitrary"),
            vmem_limit_bytes=100 * 1024 * 1024,
        ),
    )


def kernel(fire_signal, swin_2d, fire_frames, pitch_tokens, W_pitch, b_pitch, W_sw, b_sw):
    B, H, W, D = swin_2d.shape
    T = fire_signal.shape[1]
    N = fire_frames.shape[1]
    P, DP = pitch_tokens.shape[2], pitch_tokens.shape[3]
    DM = W_pitch.shape[1]

    fire_flat = fire_frames.reshape(B * N)
    sel = _sc_onehot(B, W, N, T)(fire_flat)[:, : W * N]
    sel = sel.reshape(B, W, N).astype(jnp.bfloat16)
    swin_t = jnp.transpose(swin_2d, (0, 1, 3, 2))

    return _tc_fused(B, H, W, D, N, P, DP, DM)(
        swin_t,
        sel,
        pitch_tokens,
        W_sw.astype(jnp.bfloat16),
        b_sw.reshape(1, DM),
        W_pitch,
        b_pitch.reshape(1, DM),
    )
